# Initial kernel scaffold; baseline (speedup 1.0000x reference)
import jax
import jax.numpy as jnp
from jax.experimental import pallas as pl


def kernel(items, operations, parents, item_assembly_edge_index, operation_assembly_edge_index, W_self_0, b_self_0, W_self_1, b_self_1, W_self_2, b_self_2, W_parent_0, b_parent_0, W_parent_1, b_parent_1, W_parent_2, b_parent_2, W_children_0, b_children_0, W_children_1, b_children_1, W_children_2, b_children_2, W_operations_0, b_operations_0, W_operations_1, b_operations_1, W_operations_2, b_operations_2, W_combined_0, b_combined_0, W_combined_1, b_combined_1, W_combined_2, b_combined_2):
    raise NotImplementedError("write your pallas kernel here")


# speedup vs baseline: 1.7996x; 1.7996x over previous
import functools

import jax
import jax.numpy as jnp
from jax import lax
from jax.experimental import pallas as pl
from jax.experimental.pallas import tpu as pltpu
from jax.experimental.pallas import tpu_sc as plsc

N_ITEMS = 10000
N_OPS = 50000
E = 160000

K = 128
NCHUNK = 80
SUBEDGE = K * NCHUNK
EPAD = 16 * SUBEDGE
NACC = 10112
ZROWS = NACC // 16

NPAR = 10240
PK = 80
PCHUNK = 4
PER_W = PK * PCHUNK


def _sc_body(items_hbm, items2_hbm, ops2_hbm,
             ia0_hbm, ia1_hbm, oa0_hbm, oa1_hbm,
             par_hbm, zeros_hbm,
             out_par, out_ch, out_op,
             pidx_v, prows_v, idxd_v, idxs_v, rows_v, acc, sem):
    c = lax.axis_index("c")
    s = lax.axis_index("s")
    w = c * 16 + s

    for j in range(PCHUNK):
        off = w * PER_W + j * PK
        pltpu.sync_copy(par_hbm.at[pl.ds(off, PK)], pidx_v)
        pltpu.async_copy(items_hbm.at[pidx_v], prows_v, sem).wait()
        pltpu.sync_copy(prows_v, out_par.at[pl.ds(off, PK)])

    def edge_phase(dst_hbm, src_hbm, table_hbm, out_hbm):
        pltpu.sync_copy(zeros_hbm.at[pl.ds(s * ZROWS, ZROWS)],
                        acc.at[pl.ds(s * ZROWS, ZROWS)])
        plsc.subcore_barrier()

        def chunk(j, carry):
            off = s * SUBEDGE + j * K
            pltpu.sync_copy(dst_hbm.at[pl.ds(off, K)], idxd_v)
            pltpu.sync_copy(src_hbm.at[pl.ds(c * EPAD + off, K)], idxs_v)
            pltpu.async_copy(table_hbm.at[idxs_v], rows_v, sem).wait()
            pltpu.sync_copy(rows_v, acc.at[idxd_v], add=True)
            return carry

        lax.fori_loop(0, NCHUNK, chunk, 0)
        plsc.subcore_barrier()
        pltpu.sync_copy(acc.at[pl.ds(s * ZROWS, ZROWS)],
                        out_hbm.at[pl.ds(c * NACC + s * ZROWS, ZROWS)])
        plsc.subcore_barrier()

    edge_phase(ia0_hbm, ia1_hbm, items2_hbm, out_ch)
    edge_phase(oa0_hbm, oa1_hbm, ops2_hbm, out_op)


def _sc_gather_scatter(items, items2, ops2, ia0, ia1, oa0, oa1, par, zeros):
    mesh = plsc.VectorSubcoreMesh(core_axis_name="c", subcore_axis_name="s")
    fn = functools.partial(
        pl.kernel, _sc_body, mesh=mesh,
        out_type=(
            jax.ShapeDtypeStruct((NPAR, 256), jnp.float32),
            jax.ShapeDtypeStruct((2 * NACC, 128), jnp.float32),
            jax.ShapeDtypeStruct((2 * NACC, 128), jnp.float32),
        ),
        scratch_types=[
            pltpu.VMEM((PK,), jnp.int32),
            pltpu.VMEM((PK, 256), jnp.float32),
            pltpu.VMEM((K,), jnp.int32),
            pltpu.VMEM((K,), jnp.int32),
            pltpu.VMEM((K, 128), jnp.float32),
            pltpu.VMEM_SHARED((NACC, 128), jnp.float32),
            pltpu.SemaphoreType.DMA,
        ],
    )()
    return fn(items, items2, ops2, ia0, ia1, oa0, oa1, par, zeros)


BN = 1000


def _elu(x):
    return jnp.where(x > 0, x, jnp.exp(jnp.minimum(x, 0.0)) - 1.0)


def _mlp_full_body(x_ref, w0_ref, b0_ref, w1_ref, b1_ref, w2_ref, b2_ref, o_ref):
    h = jnp.dot(x_ref[...], w0_ref[...], preferred_element_type=jnp.float32)
    h = _elu(h + b0_ref[...])
    h = jnp.dot(h, w1_ref[...], preferred_element_type=jnp.float32)
    h = _elu(h + b1_ref[...])
    o_ref[...] = jnp.dot(h, w2_ref[...], preferred_element_type=jnp.float32) + b2_ref[...]


def _mlp_full(x, w0, b0, w1, b1, w2, b2):
    n = x.shape[0]
    grid = n // BN
    return pl.pallas_call(
        _mlp_full_body,
        grid=(grid,),
        in_specs=[
            pl.BlockSpec((BN, 256), lambda i: (i, 0)),
            pl.BlockSpec((256, 512), lambda i: (0, 0)),
            pl.BlockSpec((1, 512), lambda i: (0, 0)),
            pl.BlockSpec((512, 512), lambda i: (0, 0)),
            pl.BlockSpec((1, 512), lambda i: (0, 0)),
            pl.BlockSpec((512, 256), lambda i: (0, 0)),
            pl.BlockSpec((1, 256), lambda i: (0, 0)),
        ],
        out_specs=pl.BlockSpec((BN, 256), lambda i: (i, 0)),
        out_shape=jax.ShapeDtypeStruct((n, 256), jnp.float32),
    )(x, w0, b0.reshape(1, -1), w1, b1.reshape(1, -1), w2, b2.reshape(1, -1))


def _mlp_split_body(xl_ref, xh_ref, w0a_ref, w0b_ref, b0_ref, w1_ref, b1_ref,
                    w2_ref, b2_ref, o_ref):
    h = jnp.dot(xl_ref[...], w0a_ref[...], preferred_element_type=jnp.float32)
    h += jnp.dot(xh_ref[...], w0b_ref[...], preferred_element_type=jnp.float32)
    h = _elu(h + b0_ref[...])
    h = jnp.dot(h, w1_ref[...], preferred_element_type=jnp.float32)
    h = _elu(h + b1_ref[...])
    o_ref[...] = jnp.dot(h, w2_ref[...], preferred_element_type=jnp.float32) + b2_ref[...]


def _mlp_split(xl, xh, w0, b0, w1, b1, w2, b2):
    n = xl.shape[0]
    grid = n // BN
    return pl.pallas_call(
        _mlp_split_body,
        grid=(grid,),
        in_specs=[
            pl.BlockSpec((BN, 128), lambda i: (i, 0)),
            pl.BlockSpec((BN, 128), lambda i: (i, 0)),
            pl.BlockSpec((128, 512), lambda i: (0, 0)),
            pl.BlockSpec((128, 512), lambda i: (0, 0)),
            pl.BlockSpec((1, 512), lambda i: (0, 0)),
            pl.BlockSpec((512, 512), lambda i: (0, 0)),
            pl.BlockSpec((1, 512), lambda i: (0, 0)),
            pl.BlockSpec((512, 256), lambda i: (0, 0)),
            pl.BlockSpec((1, 256), lambda i: (0, 0)),
        ],
        out_specs=pl.BlockSpec((BN, 256), lambda i: (i, 0)),
        out_shape=jax.ShapeDtypeStruct((n, 256), jnp.float32),
    )(xl, xh, w0[:128], w0[128:], b0.reshape(1, -1), w1, b1.reshape(1, -1),
      w2, b2.reshape(1, -1))


def _combined_body(xp_ref, xc_ref, xo_ref, xs_ref,
                   wp_ref, wc_ref, wo_ref, ws_ref, b0_ref,
                   w1_ref, b1_ref, w2_ref, b2_ref, o_ref):
    h = jnp.dot(xp_ref[...], wp_ref[...], preferred_element_type=jnp.float32)
    h += jnp.dot(xc_ref[...], wc_ref[...], preferred_element_type=jnp.float32)
    h += jnp.dot(xo_ref[...], wo_ref[...], preferred_element_type=jnp.float32)
    h += jnp.dot(xs_ref[...], ws_ref[...], preferred_element_type=jnp.float32)
    h = _elu(h + b0_ref[...])
    h = jnp.dot(h, w1_ref[...], preferred_element_type=jnp.float32)
    h = _elu(h + b1_ref[...])
    o = jnp.dot(h, w2_ref[...], preferred_element_type=jnp.float32) + b2_ref[...]
    i = pl.program_id(0)
    row = i * BN + lax.broadcasted_iota(jnp.int32, (BN, 256), 0)
    o_ref[...] = jnp.where(row < N_ITEMS - 1, o, 0.0)


def _combined_mlp(xp, xc, xo, xs, w0, b0, w1, b1, w2, b2):
    n = xp.shape[0]
    grid = n // BN
    xspec = pl.BlockSpec((BN, 256), lambda i: (i, 0))
    wspec = pl.BlockSpec((256, 512), lambda i: (0, 0))
    return pl.pallas_call(
        _combined_body,
        grid=(grid,),
        in_specs=[
            xspec, xspec, xspec, xspec,
            wspec, wspec, wspec, wspec,
            pl.BlockSpec((1, 512), lambda i: (0, 0)),
            pl.BlockSpec((512, 512), lambda i: (0, 0)),
            pl.BlockSpec((1, 512), lambda i: (0, 0)),
            pl.BlockSpec((512, 256), lambda i: (0, 0)),
            pl.BlockSpec((1, 256), lambda i: (0, 0)),
        ],
        out_specs=pl.BlockSpec((BN, 256), lambda i: (i, 0)),
        out_shape=jax.ShapeDtypeStruct((n, 256), jnp.float32),
    )(xp, xc, xo, xs, w0[0:256], w0[256:512], w0[512:768], w0[768:1024],
      b0.reshape(1, -1), w1, b1.reshape(1, -1), w2, b2.reshape(1, -1))


def kernel(items, operations, parents, item_assembly_edge_index, operation_assembly_edge_index, W_self_0, b_self_0, W_self_1, b_self_1, W_self_2, b_self_2, W_parent_0, b_parent_0, W_parent_1, b_parent_1, W_parent_2, b_parent_2, W_children_0, b_children_0, W_children_1, b_children_1, W_children_2, b_children_2, W_operations_0, b_operations_0, W_operations_1, b_operations_1, W_operations_2, b_operations_2, W_combined_0, b_combined_0, W_combined_1, b_combined_1, W_combined_2, b_combined_2):
    ia = item_assembly_edge_index.astype(jnp.int32)
    oa = operation_assembly_edge_index.astype(jnp.int32)
    parents = parents.astype(jnp.int32)

    pad = EPAD - E
    ia0 = jnp.concatenate([ia[0], jnp.full((pad,), N_ITEMS, jnp.int32)])
    oa0 = jnp.concatenate([oa[0], jnp.full((pad,), N_ITEMS, jnp.int32)])
    ia_s = jnp.concatenate([ia[1], jnp.zeros((pad,), jnp.int32)])
    oa_s = jnp.concatenate([oa[1], jnp.zeros((pad,), jnp.int32)])
    ia1 = jnp.concatenate([2 * ia_s, 2 * ia_s + 1])
    oa1 = jnp.concatenate([2 * oa_s, 2 * oa_s + 1])
    par = jnp.concatenate([parents, jnp.zeros((NPAR - N_ITEMS,), jnp.int32)])

    items2 = items.reshape(2 * N_ITEMS, 128)
    ops2 = operations.reshape(2 * N_OPS, 128)
    zeros = jnp.zeros((NACC, 128), jnp.float32)

    out_par, out_ch, out_op = _sc_gather_scatter(
        items, items2, ops2, ia0, ia1, oa0, oa1, par, zeros)

    ch_lo, ch_hi = out_ch[:N_ITEMS], out_ch[NACC:NACC + N_ITEMS]
    op_lo, op_hi = out_op[:N_ITEMS], out_op[NACC:NACC + N_ITEMS]

    self_emb = _mlp_full(items, W_self_0, b_self_0, W_self_1, b_self_1,
                         W_self_2, b_self_2)
    parent_emb = _mlp_full(out_par[:N_ITEMS], W_parent_0, b_parent_0,
                           W_parent_1, b_parent_1, W_parent_2, b_parent_2)
    ch_emb = _mlp_split(ch_lo, ch_hi, W_children_0, b_children_0,
                        W_children_1, b_children_1, W_children_2, b_children_2)
    op_emb = _mlp_split(op_lo, op_hi, W_operations_0, b_operations_0,
                        W_operations_1, b_operations_1, W_operations_2,
                        b_operations_2)
    return _combined_mlp(parent_emb, ch_emb, op_emb, self_emb,
                         W_combined_0, b_combined_0, W_combined_1, b_combined_1,
                         W_combined_2, b_combined_2)
